# baseline (speedup 1.0000x reference)
import functools

import jax
import jax.numpy as jnp
from jax import lax
from jax.experimental import pallas as pl
from jax.experimental.pallas import tpu as pltpu
from jax.experimental.pallas import tpu_sc as plsc

N_NODES = 10000
N_EDGES = 320000
DIM = 64
EDGE_DIM = 7
POINT_DIM = 3
N_GRAPHS = 128
TPG = 16
ACTION_DIM = 36


_SC_INFO = plsc.get_sparse_core_info()
_NC, _NS, _L = _SC_INFO.num_cores, _SC_INFO.num_subcores, _SC_INFO.num_lanes
_NW = _NC * _NS
_SC_MESH = plsc.VectorSubcoreMesh(core_axis_name="c", subcore_axis_name="s")
_SC_PARAMS = pltpu.CompilerParams(needs_layout_passes=False)

_EPW = N_EDGES // _NW


def _deg_body(dst_hbm, out_hbm, dst_v, hist_v, sem):
    wid = lax.axis_index("s") * _NC + lax.axis_index("c")
    base = wid * _EPW
    pltpu.sync_copy(dst_hbm.at[pl.ds(base, _EPW)], dst_v)
    zeros = jnp.zeros((_L,), jnp.float32)
    ones = jnp.ones((_L,), jnp.float32)

    def zero_body(i, _):
        hist_v[pl.ds(i * _L, _L)] = zeros
        return 0

    lax.fori_loop(0, N_NODES // _L, zero_body, 0)

    def acc_body(g, _):
        dv = dst_v[pl.ds(g * _L, _L)]
        plsc.addupdate_scatter(hist_v, [dv], ones)
        return 0

    lax.fori_loop(0, _EPW // _L, acc_body, 0)
    pltpu.sync_copy(hist_v, out_hbm.at[wid])


@functools.partial(jax.jit, static_argnames=())
def _sc_deg(dst):
    k = pl.kernel(
        _deg_body,
        out_type=jax.ShapeDtypeStruct((_NW, N_NODES), jnp.float32),
        mesh=_SC_MESH,
        scratch_types=[
            pltpu.VMEM((_EPW,), jnp.int32),
            pltpu.VMEM((N_NODES,), jnp.float32),
            pltpu.SemaphoreType.DMA,
        ],
        compiler_params=_SC_PARAMS,
    )
    return k(dst)


_FPW = DIM // _NW
_MSG_CH = 6400
_NCHUNK = N_EDGES // _MSG_CH
_UNROLL = 4


def _msg_body(pt_hbm, et_hbm, src_hbm, dst_hbm, out_hbm, p0, p1, a0, a1, b0, b1,
              e0A, e0B, e1A, e1B, sA, sB, dA, dB, sem):
    wid = lax.axis_index("s") * _NC + lax.axis_index("c")
    fbase = wid * _FPW
    slots = ((e0A, e1A, sA, dA), (e0B, e1B, sB, dB))

    def start_chunk(c, slot):
        e0b, e1b, sb, db = slots[slot]
        ecp0 = pltpu.async_copy(
            et_hbm.at[pl.ds(fbase * N_EDGES + c * _MSG_CH, _MSG_CH)], e0b, sem)
        ecp1 = pltpu.async_copy(
            et_hbm.at[pl.ds((fbase + 1) * N_EDGES + c * _MSG_CH, _MSG_CH)], e1b, sem)
        scp = pltpu.async_copy(src_hbm.at[pl.ds(c * _MSG_CH, _MSG_CH)], sb, sem)
        dcp = pltpu.async_copy(dst_hbm.at[pl.ds(c * _MSG_CH, _MSG_CH)], db, sem)
        return ecp0, ecp1, scp, dcp

    cps0 = start_chunk(0, 0)
    pltpu.sync_copy(pt_hbm.at[pl.ds(fbase * N_NODES, N_NODES)], p0)
    pltpu.sync_copy(pt_hbm.at[pl.ds((fbase + 1) * N_NODES, N_NODES)], p1)
    zeros = jnp.zeros((_L,), jnp.float32)

    def zero_body(i, _):
        for u in range(_UNROLL):
            a0[pl.ds((i * _UNROLL + u) * _L, _L)] = zeros
            a1[pl.ds((i * _UNROLL + u) * _L, _L)] = zeros
            b0[pl.ds((i * _UNROLL + u) * _L, _L)] = zeros
            b1[pl.ds((i * _UNROLL + u) * _L, _L)] = zeros
        return 0

    lax.fori_loop(0, N_NODES // (_L * _UNROLL), zero_body, 0)

    def do_chunk(slot):
        e0b, e1b, sb, db = slots[slot]

        def grp_body(i, _):
            for u in range(_UNROLL):
                g = i * _UNROLL + u
                sv = sb[pl.ds(g * _L, _L)]
                dv = db[pl.ds(g * _L, _L)]
                acc0 = a0 if u % 2 == 0 else b0
                acc1 = a1 if u % 2 == 0 else b1
                r0 = plsc.load_gather(p0, [sv])
                e0 = e0b[pl.ds(g * _L, _L)]
                plsc.addupdate_scatter(acc0, [dv], jnp.maximum(r0 + e0, 0.0))
                r1 = plsc.load_gather(p1, [sv])
                e1 = e1b[pl.ds(g * _L, _L)]
                plsc.addupdate_scatter(acc1, [dv], jnp.maximum(r1 + e1, 0.0))
            return 0

        lax.fori_loop(0, _MSG_CH // (_L * _UNROLL), grp_body, 0)

    cps = cps0
    for c in range(_NCHUNK):
        for cp in cps:
            cp.wait()
        if c + 1 < _NCHUNK:
            cps = start_chunk(c + 1, (c + 1) % 2)
        do_chunk(c % 2)

    def merge_body(i, _):
        for u in range(_UNROLL):
            sl = pl.ds((i * _UNROLL + u) * _L, _L)
            a0[sl] = a0[sl] + b0[sl]
            a1[sl] = a1[sl] + b1[sl]
        return 0

    lax.fori_loop(0, N_NODES // (_L * _UNROLL), merge_body, 0)
    pltpu.sync_copy(a0, out_hbm.at[pl.ds(fbase * N_NODES, N_NODES)])
    pltpu.sync_copy(a1, out_hbm.at[pl.ds((fbase + 1) * N_NODES, N_NODES)])


@jax.jit
def _sc_msg(pt_flat, et_flat, src, dst):
    k = pl.kernel(
        _msg_body,
        out_type=jax.ShapeDtypeStruct((DIM * N_NODES,), jnp.float32),
        mesh=_SC_MESH,
        scratch_types=[
            pltpu.VMEM((N_NODES,), jnp.float32),
            pltpu.VMEM((N_NODES,), jnp.float32),
            pltpu.VMEM((N_NODES,), jnp.float32),
            pltpu.VMEM((N_NODES,), jnp.float32),
            pltpu.VMEM((N_NODES,), jnp.float32),
            pltpu.VMEM((N_NODES,), jnp.float32),
            pltpu.VMEM((_MSG_CH,), jnp.float32),
            pltpu.VMEM((_MSG_CH,), jnp.float32),
            pltpu.VMEM((_MSG_CH,), jnp.float32),
            pltpu.VMEM((_MSG_CH,), jnp.float32),
            pltpu.VMEM((_MSG_CH,), jnp.int32),
            pltpu.VMEM((_MSG_CH,), jnp.int32),
            pltpu.VMEM((_MSG_CH,), jnp.int32),
            pltpu.VMEM((_MSG_CH,), jnp.int32),
            pltpu.SemaphoreType.DMA,
        ],
        compiler_params=_SC_PARAMS,
    )
    return k(pt_flat, et_flat, src, dst)


def _lstm_cell(x, h, c, Wih, Whh, bih, bhh):
    g = x @ Wih + h @ Whh + bih + bhh
    i, f, gg, o = jnp.split(g, 4, axis=-1)
    i = jax.nn.sigmoid(i)
    f = jax.nn.sigmoid(f)
    gg = jnp.tanh(gg)
    o = jax.nn.sigmoid(o)
    c2 = f * c + i * gg
    h2 = o * jnp.tanh(c2)
    return h2, c2


def _gru_cell(x, h, Wih, Whh, bih, bhh):
    gi = x @ Wih + bih
    gh = h @ Whh + bhh
    ir, iz, inn = jnp.split(gi, 3, axis=-1)
    hr, hz, hn = jnp.split(gh, 3, axis=-1)
    r = jax.nn.sigmoid(ir + hr)
    z = jax.nn.sigmoid(iz + hz)
    n = jnp.tanh(inn + r * hn)
    return (1.0 - z) * n + z * h


def _mpnn(p, x, src, dst, edge_attr):
    out = jax.nn.relu(x @ p['lin0_W'] + p['lin0_b'])
    e = jax.nn.relu(edge_attr @ p['e_W1'] + p['e_b1']) @ p['e_W2'] + p['e_b2']
    deg = jnp.sum(_sc_deg(dst), axis=0)
    deg = jnp.maximum(deg, 1.0)[:, None]
    eT = e.T.reshape(-1)
    h = out
    for _ in range(6):
        pt = (out @ p['m_W'] + p['m_b']).T.reshape(-1)
        agg = _sc_msg(pt, eT, src, dst).reshape(DIM, N_NODES).T / deg
        h = _gru_cell(agg, h, p['g_Wih'], p['g_Whh'], p['g_bih'], p['g_bhh'])
        out = h
    return out


def _set2set(p, out, batch):
    h = jnp.zeros((N_GRAPHS, DIM), jnp.float32)
    c = jnp.zeros((N_GRAPHS, DIM), jnp.float32)
    q_star = jnp.zeros((N_GRAPHS, 2 * DIM), jnp.float32)
    for _ in range(6):
        h, c = _lstm_cell(q_star, h, c, p['Wih'], p['Whh'], p['bih'], p['bhh'])
        e = jnp.sum(out * h[batch], axis=-1)
        emax = jax.ops.segment_max(e, batch, num_segments=N_GRAPHS)
        ex = jnp.exp(e - emax[batch])
        den = jax.ops.segment_sum(ex, batch, num_segments=N_GRAPHS)
        a = ex / (den[batch] + 1e-16)
        r = jax.ops.segment_sum(a[:, None] * out, batch, num_segments=N_GRAPHS)
        q_star = jnp.concatenate([h, r], axis=-1)
    return q_star


def kernel(x, edge_attr, actor_params, critic_params, edge_index, batch, nonring, nrbidx):
    src = edge_index[0]
    dst = edge_index[1]
    h0 = jnp.zeros((N_GRAPHS, DIM), jnp.float32)
    c0 = jnp.zeros((N_GRAPHS, DIM), jnp.float32)
    out_a = _mpnn(actor_params['mpnn'], x, src, dst, edge_attr)
    pool_a = _set2set(actor_params['s2s'], out_a, batch)
    mp = actor_params['mem']
    hp, cp = _lstm_cell(pool_a, h0, c0, mp['Wih'], mp['Whh'], mp['bih'], mp['bhh'])
    lstm_sel = hp[nrbidx]
    gath = out_a[nonring.reshape(-1)].reshape(-1, 4 * DIM)
    cat = jnp.concatenate([lstm_sel, gath], axis=1)
    ap = actor_params['mlp']
    logits = (jax.nn.relu(cat @ ap['W1'] + ap['b1']) @ ap['W2'] + ap['b2']).reshape(N_GRAPHS, TPG, ACTION_DIM)
    out_c = _mpnn(critic_params['mpnn'], x, src, dst, edge_attr)
    pool_c = _set2set(critic_params['s2s'], out_c, batch)
    mc = critic_params['mem']
    hv, cv = _lstm_cell(pool_c, h0, c0, mc['Wih'], mc['Whh'], mc['bih'], mc['bhh'])
    cpp = critic_params['mlp']
    v = jax.nn.relu(hv @ cpp['W1'] + cpp['b1']) @ cpp['W2'] + cpp['b2']
    gnoise = jax.random.gumbel(jax.random.key(1234), (N_GRAPHS, TPG, ACTION_DIM), jnp.float32)
    logp_all = jax.nn.log_softmax(logits, axis=-1)
    action = jnp.argmax(gnoise + logits, axis=-1)
    log_prob = jnp.take_along_axis(logp_all, action[..., None], axis=-1)[..., 0]
    entropy = -jnp.sum(jnp.exp(logp_all) * logp_all, axis=-1)
    return (action, log_prob, entropy, v, hp, cp, hv, cv)


# speedup vs baseline: 1.0001x; 1.0001x over previous
import functools

import jax
import jax.numpy as jnp
from jax import lax
from jax.experimental import pallas as pl
from jax.experimental.pallas import tpu as pltpu
from jax.experimental.pallas import tpu_sc as plsc

N_NODES = 10000
N_EDGES = 320000
DIM = 64
EDGE_DIM = 7
POINT_DIM = 3
N_GRAPHS = 128
TPG = 16
ACTION_DIM = 36


_SC_INFO = plsc.get_sparse_core_info()
_NC, _NS, _L = _SC_INFO.num_cores, _SC_INFO.num_subcores, _SC_INFO.num_lanes
_NW = _NC * _NS
_SC_MESH = plsc.VectorSubcoreMesh(core_axis_name="c", subcore_axis_name="s")
_SC_PARAMS = pltpu.CompilerParams(needs_layout_passes=False)

_EPW = N_EDGES // _NW


def _deg_body(dst_hbm, out_hbm, dst_v, hist_v, sem):
    wid = lax.axis_index("s") * _NC + lax.axis_index("c")
    base = wid * _EPW
    pltpu.sync_copy(dst_hbm.at[pl.ds(base, _EPW)], dst_v)
    zeros = jnp.zeros((_L,), jnp.float32)
    ones = jnp.ones((_L,), jnp.float32)

    def zero_body(i, _):
        hist_v[pl.ds(i * _L, _L)] = zeros
        return 0

    lax.fori_loop(0, N_NODES // _L, zero_body, 0)

    def acc_body(g, _):
        dv = dst_v[pl.ds(g * _L, _L)]
        plsc.addupdate_scatter(hist_v, [dv], ones)
        return 0

    lax.fori_loop(0, _EPW // _L, acc_body, 0)
    pltpu.sync_copy(hist_v, out_hbm.at[wid])


@functools.partial(jax.jit, static_argnames=())
def _sc_deg(dst):
    k = pl.kernel(
        _deg_body,
        out_type=jax.ShapeDtypeStruct((_NW, N_NODES), jnp.float32),
        mesh=_SC_MESH,
        scratch_types=[
            pltpu.VMEM((_EPW,), jnp.int32),
            pltpu.VMEM((N_NODES,), jnp.float32),
            pltpu.SemaphoreType.DMA,
        ],
        compiler_params=_SC_PARAMS,
    )
    return k(dst)


_FPW = DIM // _NW
_MSG_CH = 8000
_NCHUNK = N_EDGES // _MSG_CH
_UNROLL = 5


def _msg_body(pt_hbm, et_hbm, src_hbm, dst_hbm, out_hbm, p0, p1, a0, a1,
              e0A, e0B, e1A, e1B, sA, sB, dA, dB, sem):
    wid = lax.axis_index("s") * _NC + lax.axis_index("c")
    fbase = wid * _FPW
    slots = ((e0A, e1A, sA, dA), (e0B, e1B, sB, dB))

    def start_chunk(c, slot):
        e0b, e1b, sb, db = slots[slot]
        ecp0 = pltpu.async_copy(
            et_hbm.at[pl.ds(fbase * N_EDGES + c * _MSG_CH, _MSG_CH)], e0b, sem)
        ecp1 = pltpu.async_copy(
            et_hbm.at[pl.ds((fbase + 1) * N_EDGES + c * _MSG_CH, _MSG_CH)], e1b, sem)
        scp = pltpu.async_copy(src_hbm.at[pl.ds(c * _MSG_CH, _MSG_CH)], sb, sem)
        dcp = pltpu.async_copy(dst_hbm.at[pl.ds(c * _MSG_CH, _MSG_CH)], db, sem)
        return ecp0, ecp1, scp, dcp

    cps0 = start_chunk(0, 0)
    pltpu.sync_copy(pt_hbm.at[pl.ds(fbase * N_NODES, N_NODES)], p0)
    pltpu.sync_copy(pt_hbm.at[pl.ds((fbase + 1) * N_NODES, N_NODES)], p1)
    zeros = jnp.zeros((_L,), jnp.float32)

    def zero_body(i, _):
        for u in range(_UNROLL):
            a0[pl.ds((i * _UNROLL + u) * _L, _L)] = zeros
            a1[pl.ds((i * _UNROLL + u) * _L, _L)] = zeros
        return 0

    lax.fori_loop(0, N_NODES // (_L * _UNROLL), zero_body, 0)

    def do_chunk(slot):
        e0b, e1b, sb, db = slots[slot]

        def grp_body(i, _):
            for u in range(_UNROLL):
                g = i * _UNROLL + u
                sv = sb[pl.ds(g * _L, _L)]
                dv = db[pl.ds(g * _L, _L)]
                r0 = plsc.load_gather(p0, [sv])
                e0 = e0b[pl.ds(g * _L, _L)]
                plsc.addupdate_scatter(a0, [dv], jnp.maximum(r0 + e0, 0.0))
                r1 = plsc.load_gather(p1, [sv])
                e1 = e1b[pl.ds(g * _L, _L)]
                plsc.addupdate_scatter(a1, [dv], jnp.maximum(r1 + e1, 0.0))
            return 0

        lax.fori_loop(0, _MSG_CH // (_L * _UNROLL), grp_body, 0)

    cps = cps0
    for c in range(_NCHUNK):
        for cp in cps:
            cp.wait()
        if c + 1 < _NCHUNK:
            cps = start_chunk(c + 1, (c + 1) % 2)
        do_chunk(c % 2)

    pltpu.sync_copy(a0, out_hbm.at[pl.ds(fbase * N_NODES, N_NODES)])
    pltpu.sync_copy(a1, out_hbm.at[pl.ds((fbase + 1) * N_NODES, N_NODES)])


@jax.jit
def _sc_msg(pt_flat, et_flat, src, dst):
    k = pl.kernel(
        _msg_body,
        out_type=jax.ShapeDtypeStruct((DIM * N_NODES,), jnp.float32),
        mesh=_SC_MESH,
        scratch_types=[
            pltpu.VMEM((N_NODES,), jnp.float32),
            pltpu.VMEM((N_NODES,), jnp.float32),
            pltpu.VMEM((N_NODES,), jnp.float32),
            pltpu.VMEM((N_NODES,), jnp.float32),
            pltpu.VMEM((_MSG_CH,), jnp.float32),
            pltpu.VMEM((_MSG_CH,), jnp.float32),
            pltpu.VMEM((_MSG_CH,), jnp.float32),
            pltpu.VMEM((_MSG_CH,), jnp.float32),
            pltpu.VMEM((_MSG_CH,), jnp.int32),
            pltpu.VMEM((_MSG_CH,), jnp.int32),
            pltpu.VMEM((_MSG_CH,), jnp.int32),
            pltpu.VMEM((_MSG_CH,), jnp.int32),
            pltpu.SemaphoreType.DMA,
        ],
        compiler_params=_SC_PARAMS,
    )
    return k(pt_flat, et_flat, src, dst)


def _init_body(xt_ref, w0t_ref, b0_ref, mwt_ref, mb_ref, degp_ref,
               out0_ref, pt0_ref, deg_ref):
    out0 = jnp.maximum(jnp.dot(w0t_ref[...], xt_ref[...],
                               preferred_element_type=jnp.float32) + b0_ref[...], 0.0)
    out0_ref[...] = out0
    pt0_ref[...] = jnp.dot(mwt_ref[...], out0,
                           preferred_element_type=jnp.float32) + mb_ref[...]
    deg_ref[...] = jnp.maximum(jnp.sum(degp_ref[...], axis=0, keepdims=True), 1.0)


@jax.jit
def _tc_init(xt, w0t, b0, mwt, mb, degp):
    return pl.pallas_call(
        _init_body,
        out_shape=(
            jax.ShapeDtypeStruct((DIM, N_NODES), jnp.float32),
            jax.ShapeDtypeStruct((DIM, N_NODES), jnp.float32),
            jax.ShapeDtypeStruct((1, N_NODES), jnp.float32),
        ),
    )(xt, w0t, b0, mwt, mb, degp)


_GRU_BLK = 2500


def _gru_body(aggt_ref, ht_ref, deg_ref, wiht_ref, whht_ref, bih_ref, bhh_ref,
              mwt_ref, mb_ref, h2_ref, pt2_ref):
    aggn = aggt_ref[...] / deg_ref[...]
    gi = jnp.dot(wiht_ref[...], aggn, preferred_element_type=jnp.float32) + bih_ref[...]
    gh = jnp.dot(whht_ref[...], ht_ref[...], preferred_element_type=jnp.float32) + bhh_ref[...]
    r = jax.nn.sigmoid(gi[0:DIM] + gh[0:DIM])
    z = jax.nn.sigmoid(gi[DIM:2 * DIM] + gh[DIM:2 * DIM])
    n = jnp.tanh(gi[2 * DIM:3 * DIM] + r * gh[2 * DIM:3 * DIM])
    h2 = (1.0 - z) * n + z * ht_ref[...]
    h2_ref[...] = h2
    pt2_ref[...] = jnp.dot(mwt_ref[...], h2, preferred_element_type=jnp.float32) + mb_ref[...]


@jax.jit
def _tc_gru(aggt, ht, deg, wiht, whht, bih, bhh, mwt, mb):
    return pl.pallas_call(
        _gru_body,
        out_shape=(
            jax.ShapeDtypeStruct((DIM, N_NODES), jnp.float32),
            jax.ShapeDtypeStruct((DIM, N_NODES), jnp.float32),
        ),
    )(aggt, ht, deg, wiht, whht, bih, bhh, mwt, mb)


_EMLP_BLK = 3200


def _emlp_body(eat_ref, w1t_ref, b1_ref, w2t_ref, b2_ref, et_ref):
    h1 = jnp.maximum(jnp.dot(w1t_ref[...], eat_ref[...],
                             preferred_element_type=jnp.float32) + b1_ref[...], 0.0)
    et_ref[...] = jnp.dot(w2t_ref[...], h1,
                          preferred_element_type=jnp.float32) + b2_ref[...]


@jax.jit
def _tc_emlp(eat, w1t, b1, w2t, b2):
    nb = N_EDGES // _EMLP_BLK
    full = lambda s: pl.BlockSpec(s, lambda i: (0, 0))
    col = lambda r: pl.BlockSpec((r, _EMLP_BLK), lambda i: (0, i))
    return pl.pallas_call(
        _emlp_body,
        grid=(nb,),
        in_specs=[col(8), full((DIM, 8)), full((DIM, 1)),
                  full((DIM, DIM)), full((DIM, 1))],
        out_specs=col(DIM),
        out_shape=jax.ShapeDtypeStruct((DIM, N_EDGES), jnp.float32),
    )(eat, w1t, b1, w2t, b2)


def _lstm_cell(x, h, c, Wih, Whh, bih, bhh):
    g = x @ Wih + h @ Whh + bih + bhh
    i, f, gg, o = jnp.split(g, 4, axis=-1)
    i = jax.nn.sigmoid(i)
    f = jax.nn.sigmoid(f)
    gg = jnp.tanh(gg)
    o = jax.nn.sigmoid(o)
    c2 = f * c + i * gg
    h2 = o * jnp.tanh(c2)
    return h2, c2


def _gru_cell(x, h, Wih, Whh, bih, bhh):
    gi = x @ Wih + bih
    gh = h @ Whh + bhh
    ir, iz, inn = jnp.split(gi, 3, axis=-1)
    hr, hz, hn = jnp.split(gh, 3, axis=-1)
    r = jax.nn.sigmoid(ir + hr)
    z = jax.nn.sigmoid(iz + hz)
    n = jnp.tanh(inn + r * hn)
    return (1.0 - z) * n + z * h


def _mpnn(p, xt_pad, src, dst, eat_pad, degp):
    w0t = jnp.zeros((DIM, 8), jnp.float32).at[:, 0:POINT_DIM].set(p['lin0_W'].T)
    b0 = p['lin0_b'][:, None]
    w1t = jnp.zeros((DIM, 8), jnp.float32).at[:, 0:EDGE_DIM].set(p['e_W1'].T)
    b1 = p['e_b1'][:, None]
    w2t = p['e_W2'].T
    b2 = p['e_b2'][:, None]
    mwt = p['m_W'].T
    mb = p['m_b'][:, None]
    wiht = p['g_Wih'].T
    whht = p['g_Whh'].T
    bih = p['g_bih'][:, None]
    bhh = p['g_bhh'][:, None]

    out0t, pt, deg = _tc_init(xt_pad, w0t, b0, mwt, mb, degp)
    et_flat = _tc_emlp(eat_pad, w1t, b1, w2t, b2).reshape(-1)
    ht = out0t
    for _ in range(6):
        agg = _sc_msg(pt.reshape(-1), et_flat, src, dst).reshape(DIM, N_NODES)
        ht, pt = _tc_gru(agg, ht, deg, wiht, whht, bih, bhh, mwt, mb)
    return ht


def _set2set(p, out, batch):
    h = jnp.zeros((N_GRAPHS, DIM), jnp.float32)
    c = jnp.zeros((N_GRAPHS, DIM), jnp.float32)
    q_star = jnp.zeros((N_GRAPHS, 2 * DIM), jnp.float32)
    for _ in range(6):
        h, c = _lstm_cell(q_star, h, c, p['Wih'], p['Whh'], p['bih'], p['bhh'])
        e = jnp.sum(out * h[batch], axis=-1)
        emax = jax.ops.segment_max(e, batch, num_segments=N_GRAPHS)
        ex = jnp.exp(e - emax[batch])
        den = jax.ops.segment_sum(ex, batch, num_segments=N_GRAPHS)
        a = ex / (den[batch] + 1e-16)
        r = jax.ops.segment_sum(a[:, None] * out, batch, num_segments=N_GRAPHS)
        q_star = jnp.concatenate([h, r], axis=-1)
    return q_star


def kernel(x, edge_attr, actor_params, critic_params, edge_index, batch, nonring, nrbidx):
    src = edge_index[0]
    dst = edge_index[1]
    h0 = jnp.zeros((N_GRAPHS, DIM), jnp.float32)
    c0 = jnp.zeros((N_GRAPHS, DIM), jnp.float32)
    xt_pad = jnp.zeros((8, N_NODES), jnp.float32).at[0:POINT_DIM].set(x.T)
    eat_pad = jnp.zeros((8, N_EDGES), jnp.float32).at[0:EDGE_DIM].set(edge_attr.T)
    degp = _sc_deg(dst)
    out_a = _mpnn(actor_params['mpnn'], xt_pad, src, dst, eat_pad, degp).T
    pool_a = _set2set(actor_params['s2s'], out_a, batch)
    mp = actor_params['mem']
    hp, cp = _lstm_cell(pool_a, h0, c0, mp['Wih'], mp['Whh'], mp['bih'], mp['bhh'])
    lstm_sel = hp[nrbidx]
    gath = out_a[nonring.reshape(-1)].reshape(-1, 4 * DIM)
    cat = jnp.concatenate([lstm_sel, gath], axis=1)
    ap = actor_params['mlp']
    logits = (jax.nn.relu(cat @ ap['W1'] + ap['b1']) @ ap['W2'] + ap['b2']).reshape(N_GRAPHS, TPG, ACTION_DIM)
    out_c = _mpnn(critic_params['mpnn'], xt_pad, src, dst, eat_pad, degp).T
    pool_c = _set2set(critic_params['s2s'], out_c, batch)
    mc = critic_params['mem']
    hv, cv = _lstm_cell(pool_c, h0, c0, mc['Wih'], mc['Whh'], mc['bih'], mc['bhh'])
    cpp = critic_params['mlp']
    v = jax.nn.relu(hv @ cpp['W1'] + cpp['b1']) @ cpp['W2'] + cpp['b2']
    gnoise = jax.random.gumbel(jax.random.key(1234), (N_GRAPHS, TPG, ACTION_DIM), jnp.float32)
    logp_all = jax.nn.log_softmax(logits, axis=-1)
    action = jnp.argmax(gnoise + logits, axis=-1)
    log_prob = jnp.take_along_axis(logp_all, action[..., None], axis=-1)[..., 0]
    entropy = -jnp.sum(jnp.exp(logp_all) * logp_all, axis=-1)
    return (action, log_prob, entropy, v, hp, cp, hv, cv)


# speedup vs baseline: 1.4933x; 1.4931x over previous
import functools

import jax
import jax.numpy as jnp
from jax import lax
from jax.experimental import pallas as pl
from jax.experimental.pallas import tpu as pltpu
from jax.experimental.pallas import tpu_sc as plsc

N_NODES = 10000
N_EDGES = 320000
DIM = 64
EDGE_DIM = 7
POINT_DIM = 3
N_GRAPHS = 128
TPG = 16
ACTION_DIM = 36


_SC_INFO = plsc.get_sparse_core_info()
_NC, _NS, _L = _SC_INFO.num_cores, _SC_INFO.num_subcores, _SC_INFO.num_lanes
_NW = _NC * _NS
_SC_MESH = plsc.VectorSubcoreMesh(core_axis_name="c", subcore_axis_name="s")
_SC_PARAMS = pltpu.CompilerParams(needs_layout_passes=False)

_EPW = N_EDGES // _NW


def _deg_body(dst_hbm, out_hbm, dst_v, hist_v, sem):
    wid = lax.axis_index("s") * _NC + lax.axis_index("c")
    base = wid * _EPW
    pltpu.sync_copy(dst_hbm.at[pl.ds(base, _EPW)], dst_v)
    zeros = jnp.zeros((_L,), jnp.float32)
    ones = jnp.ones((_L,), jnp.float32)

    def zero_body(i, _):
        hist_v[pl.ds(i * _L, _L)] = zeros
        return 0

    lax.fori_loop(0, N_NODES // _L, zero_body, 0)

    def acc_body(g, _):
        dv = dst_v[pl.ds(g * _L, _L)]
        plsc.addupdate_scatter(hist_v, [dv], ones)
        return 0

    lax.fori_loop(0, _EPW // _L, acc_body, 0)
    pltpu.sync_copy(hist_v, out_hbm.at[wid])


@functools.partial(jax.jit, static_argnames=())
def _sc_deg(dst):
    k = pl.kernel(
        _deg_body,
        out_type=jax.ShapeDtypeStruct((_NW, N_NODES), jnp.float32),
        mesh=_SC_MESH,
        scratch_types=[
            pltpu.VMEM((_EPW,), jnp.int32),
            pltpu.VMEM((N_NODES,), jnp.float32),
            pltpu.SemaphoreType.DMA,
        ],
        compiler_params=_SC_PARAMS,
    )
    return k(dst)


_FPW = DIM // _NW
_MSG_CH = 8000
_NCHUNK = N_EDGES // _MSG_CH
_UNROLL = 5


def _msg_body(pt_hbm, et_hbm, src_hbm, dst_hbm, out_hbm, p0, p1, a0, a1,
              e0A, e0B, e1A, e1B, sA, sB, dA, dB, sem):
    wid = lax.axis_index("s") * _NC + lax.axis_index("c")
    fbase = wid * _FPW
    slots = ((e0A, e1A, sA, dA), (e0B, e1B, sB, dB))

    def start_chunk(c, slot):
        e0b, e1b, sb, db = slots[slot]
        ecp0 = pltpu.async_copy(
            et_hbm.at[pl.ds(fbase * N_EDGES + c * _MSG_CH, _MSG_CH)], e0b, sem)
        ecp1 = pltpu.async_copy(
            et_hbm.at[pl.ds((fbase + 1) * N_EDGES + c * _MSG_CH, _MSG_CH)], e1b, sem)
        scp = pltpu.async_copy(src_hbm.at[pl.ds(c * _MSG_CH, _MSG_CH)], sb, sem)
        dcp = pltpu.async_copy(dst_hbm.at[pl.ds(c * _MSG_CH, _MSG_CH)], db, sem)
        return ecp0, ecp1, scp, dcp

    cps0 = start_chunk(0, 0)
    pltpu.sync_copy(pt_hbm.at[pl.ds(fbase * N_NODES, N_NODES)], p0)
    pltpu.sync_copy(pt_hbm.at[pl.ds((fbase + 1) * N_NODES, N_NODES)], p1)
    zeros = jnp.zeros((_L,), jnp.float32)

    def zero_body(i, _):
        for u in range(_UNROLL):
            a0[pl.ds((i * _UNROLL + u) * _L, _L)] = zeros
            a1[pl.ds((i * _UNROLL + u) * _L, _L)] = zeros
        return 0

    lax.fori_loop(0, N_NODES // (_L * _UNROLL), zero_body, 0)

    def do_chunk(slot):
        e0b, e1b, sb, db = slots[slot]

        def grp_body(i, _):
            for u in range(_UNROLL):
                g = i * _UNROLL + u
                sv = sb[pl.ds(g * _L, _L)]
                dv = db[pl.ds(g * _L, _L)]
                r0 = plsc.load_gather(p0, [sv])
                e0 = e0b[pl.ds(g * _L, _L)]
                plsc.addupdate_scatter(a0, [dv], jnp.maximum(r0 + e0, 0.0))
                r1 = plsc.load_gather(p1, [sv])
                e1 = e1b[pl.ds(g * _L, _L)]
                plsc.addupdate_scatter(a1, [dv], jnp.maximum(r1 + e1, 0.0))
            return 0

        lax.fori_loop(0, _MSG_CH // (_L * _UNROLL), grp_body, 0)

    cps = cps0
    for c in range(_NCHUNK):
        for cp in cps:
            cp.wait()
        if c + 1 < _NCHUNK:
            cps = start_chunk(c + 1, (c + 1) % 2)
        do_chunk(c % 2)

    pltpu.sync_copy(a0, out_hbm.at[pl.ds(fbase * N_NODES, N_NODES)])
    pltpu.sync_copy(a1, out_hbm.at[pl.ds((fbase + 1) * N_NODES, N_NODES)])


@jax.jit
def _sc_msg(pt_flat, et_flat, src, dst):
    k = pl.kernel(
        _msg_body,
        out_type=jax.ShapeDtypeStruct((DIM * N_NODES,), jnp.float32),
        mesh=_SC_MESH,
        scratch_types=[
            pltpu.VMEM((N_NODES,), jnp.float32),
            pltpu.VMEM((N_NODES,), jnp.float32),
            pltpu.VMEM((N_NODES,), jnp.float32),
            pltpu.VMEM((N_NODES,), jnp.float32),
            pltpu.VMEM((_MSG_CH,), jnp.float32),
            pltpu.VMEM((_MSG_CH,), jnp.float32),
            pltpu.VMEM((_MSG_CH,), jnp.float32),
            pltpu.VMEM((_MSG_CH,), jnp.float32),
            pltpu.VMEM((_MSG_CH,), jnp.int32),
            pltpu.VMEM((_MSG_CH,), jnp.int32),
            pltpu.VMEM((_MSG_CH,), jnp.int32),
            pltpu.VMEM((_MSG_CH,), jnp.int32),
            pltpu.SemaphoreType.DMA,
        ],
        compiler_params=_SC_PARAMS,
    )
    return k(pt_flat, et_flat, src, dst)


def _init_body(xt_ref, w0t_ref, b0_ref, mwt_ref, mb_ref, degp_ref,
               out0_ref, pt0_ref, deg_ref):
    out0 = jnp.maximum(jnp.dot(w0t_ref[...], xt_ref[...],
                               preferred_element_type=jnp.float32) + b0_ref[...], 0.0)
    out0_ref[...] = out0
    pt0_ref[...] = jnp.dot(mwt_ref[...], out0,
                           preferred_element_type=jnp.float32) + mb_ref[...]
    deg_ref[...] = jnp.maximum(jnp.sum(degp_ref[...], axis=0, keepdims=True), 1.0)


@jax.jit
def _tc_init(xt, w0t, b0, mwt, mb, degp):
    return pl.pallas_call(
        _init_body,
        out_shape=(
            jax.ShapeDtypeStruct((DIM, N_NODES), jnp.float32),
            jax.ShapeDtypeStruct((DIM, N_NODES), jnp.float32),
            jax.ShapeDtypeStruct((1, N_NODES), jnp.float32),
        ),
    )(xt, w0t, b0, mwt, mb, degp)


_GRU_BLK = 2500


def _gru_body(aggt_ref, ht_ref, deg_ref, wiht_ref, whht_ref, bih_ref, bhh_ref,
              mwt_ref, mb_ref, h2_ref, pt2_ref):
    aggn = aggt_ref[...] / deg_ref[...]
    gi = jnp.dot(wiht_ref[...], aggn, preferred_element_type=jnp.float32) + bih_ref[...]
    gh = jnp.dot(whht_ref[...], ht_ref[...], preferred_element_type=jnp.float32) + bhh_ref[...]
    r = jax.nn.sigmoid(gi[0:DIM] + gh[0:DIM])
    z = jax.nn.sigmoid(gi[DIM:2 * DIM] + gh[DIM:2 * DIM])
    n = jnp.tanh(gi[2 * DIM:3 * DIM] + r * gh[2 * DIM:3 * DIM])
    h2 = (1.0 - z) * n + z * ht_ref[...]
    h2_ref[...] = h2
    pt2_ref[...] = jnp.dot(mwt_ref[...], h2, preferred_element_type=jnp.float32) + mb_ref[...]


@jax.jit
def _tc_gru(aggt, ht, deg, wiht, whht, bih, bhh, mwt, mb):
    return pl.pallas_call(
        _gru_body,
        out_shape=(
            jax.ShapeDtypeStruct((DIM, N_NODES), jnp.float32),
            jax.ShapeDtypeStruct((DIM, N_NODES), jnp.float32),
        ),
    )(aggt, ht, deg, wiht, whht, bih, bhh, mwt, mb)


_EMLP_BLK = 3200


def _emlp_body(eat_ref, w1t_ref, b1_ref, w2t_ref, b2_ref, et_ref):
    h1 = jnp.maximum(jnp.dot(w1t_ref[...], eat_ref[...],
                             preferred_element_type=jnp.float32) + b1_ref[...], 0.0)
    et_ref[...] = jnp.dot(w2t_ref[...], h1,
                          preferred_element_type=jnp.float32) + b2_ref[...]


@jax.jit
def _tc_emlp(eat, w1t, b1, w2t, b2):
    nb = N_EDGES // _EMLP_BLK
    full = lambda s: pl.BlockSpec(s, lambda i: (0, 0))
    col = lambda r: pl.BlockSpec((r, _EMLP_BLK), lambda i: (0, i))
    return pl.pallas_call(
        _emlp_body,
        grid=(nb,),
        in_specs=[col(8), full((DIM, 8)), full((DIM, 1)),
                  full((DIM, DIM)), full((DIM, 1))],
        out_specs=col(DIM),
        out_shape=jax.ShapeDtypeStruct((DIM, N_EDGES), jnp.float32),
    )(eat, w1t, b1, w2t, b2)


def _s2s_body(outt_ref, batch_ref, wih_ref, whh_ref, bih_ref, bhh_ref,
              wihm_ref, bihm_ref, bhhm_ref, hp_ref, cp_ref):
    outt = outt_ref[...]
    bt = batch_ref[...]
    giota = lax.broadcasted_iota(jnp.int32, (N_GRAPHS, 1), 0)
    B = (bt == giota).astype(jnp.float32)
    wih = wih_ref[...]
    whh = whh_ref[...]
    bih = bih_ref[...]
    bhh = bhh_ref[...]
    ht = jnp.zeros((DIM, N_GRAPHS), jnp.float32)
    ct = jnp.zeros((DIM, N_GRAPHS), jnp.float32)
    qt = jnp.zeros((2 * DIM, N_GRAPHS), jnp.float32)
    for _ in range(6):
        g = jnp.dot(wih, qt, preferred_element_type=jnp.float32) \
            + jnp.dot(whh, ht, preferred_element_type=jnp.float32) + bih + bhh
        ii = jax.nn.sigmoid(g[0:DIM])
        ff = jax.nn.sigmoid(g[DIM:2 * DIM])
        gg = jnp.tanh(g[2 * DIM:3 * DIM])
        oo = jax.nn.sigmoid(g[3 * DIM:4 * DIM])
        ct = ff * ct + ii * gg
        ht = oo * jnp.tanh(ct)
        hn = jnp.dot(ht, B, preferred_element_type=jnp.float32)
        e = jnp.sum(outt * hn, axis=0, keepdims=True)
        emax = jnp.max(jnp.where(B > 0.0, e, -1e30), axis=1, keepdims=True)
        emax_n = jnp.dot(emax.reshape(1, N_GRAPHS), B,
                         preferred_element_type=jnp.float32)
        ex = jnp.exp(e - emax_n)
        den = jnp.sum(B * ex, axis=1, keepdims=True)
        den_n = jnp.dot(den.reshape(1, N_GRAPHS), B,
                        preferred_element_type=jnp.float32)
        a = ex / (den_n + 1e-16)
        rt = jnp.dot(outt * a, B.T, preferred_element_type=jnp.float32)
        qt = jnp.concatenate([ht, rt], axis=0)
    gm = jnp.dot(wihm_ref[...], qt, preferred_element_type=jnp.float32) \
        + bihm_ref[...] + bhhm_ref[...]
    im = jax.nn.sigmoid(gm[0:DIM])
    gm2 = jnp.tanh(gm[2 * DIM:3 * DIM])
    om = jax.nn.sigmoid(gm[3 * DIM:4 * DIM])
    cpt = im * gm2
    hp_ref[...] = om * jnp.tanh(cpt)
    cp_ref[...] = cpt


@jax.jit
def _tc_s2s(outt, batch_row, wihT, whhT, bih, bhh, wihmT, bihm, bhhm):
    return pl.pallas_call(
        _s2s_body,
        out_shape=(
            jax.ShapeDtypeStruct((DIM, N_GRAPHS), jnp.float32),
            jax.ShapeDtypeStruct((DIM, N_GRAPHS), jnp.float32),
        ),
    )(outt, batch_row, wihT, whhT, bih, bhh, wihmT, bihm, bhhm)


_NTOR = N_GRAPHS * TPG


def _head_body(hpt_ref, gath_ref, w1_ref, b1_ref, w2_ref, b2_ref,
               hvt_ref, w1c_ref, b1c_ref, w2c_ref, b2c_ref, gn_ref,
               act_ref, lp_ref, ent_ref, v_ref):
    giota = lax.broadcasted_iota(jnp.int32, (N_GRAPHS, 1), 0)
    jg = lax.broadcasted_iota(jnp.int32, (1, _NTOR), 1) // TPG
    R = (jg == giota).astype(jnp.float32)
    sel = jnp.dot(hpt_ref[...], R, preferred_element_type=jnp.float32)
    cat = jnp.concatenate([sel, gath_ref[...]], axis=0)
    h1 = jnp.maximum(jnp.dot(w1_ref[...], cat,
                             preferred_element_type=jnp.float32) + b1_ref[...], 0.0)
    logits = jnp.dot(w2_ref[...], h1,
                     preferred_element_type=jnp.float32) + b2_ref[...]
    m = jnp.max(logits, axis=0, keepdims=True)
    sh = logits - m
    lse = jnp.log(jnp.sum(jnp.exp(sh), axis=0, keepdims=True))
    logp = sh - lse
    per = gn_ref[...] + logits
    pm = jnp.max(per, axis=0, keepdims=True)
    riota = lax.broadcasted_iota(jnp.int32, (ACTION_DIM, _NTOR), 0)
    act = jnp.min(jnp.where(per == pm, riota, ACTION_DIM), axis=0, keepdims=True)
    act_ref[...] = act
    lp_ref[...] = jnp.sum(jnp.where(riota == act, logp, 0.0), axis=0, keepdims=True)
    ent_ref[...] = -jnp.sum(jnp.exp(logp) * logp, axis=0, keepdims=True)
    h1c = jnp.maximum(jnp.dot(w1c_ref[...], hvt_ref[...],
                              preferred_element_type=jnp.float32) + b1c_ref[...], 0.0)
    v_ref[...] = jnp.dot(w2c_ref[...], h1c,
                         preferred_element_type=jnp.float32) + b2c_ref[...]


@jax.jit
def _tc_head(hpt, gath, w1, b1, w2, b2, hvt, w1c, b1c, w2c, b2c, gn):
    return pl.pallas_call(
        _head_body,
        out_shape=(
            jax.ShapeDtypeStruct((1, _NTOR), jnp.int32),
            jax.ShapeDtypeStruct((1, _NTOR), jnp.float32),
            jax.ShapeDtypeStruct((1, _NTOR), jnp.float32),
            jax.ShapeDtypeStruct((1, N_GRAPHS), jnp.float32),
        ),
    )(hpt, gath, w1, b1, w2, b2, hvt, w1c, b1c, w2c, b2c, gn)


def _gat_body(pa_hbm, nr_hbm, out_hbm, q0, q1, idx_buf, ob0, ob1, sem):
    wid = lax.axis_index("s") * _NC + lax.axis_index("c")
    fbase = wid * _FPW
    pltpu.sync_copy(pa_hbm.at[pl.ds(fbase * N_NODES, N_NODES)], q0)
    pltpu.sync_copy(pa_hbm.at[pl.ds((fbase + 1) * N_NODES, N_NODES)], q1)
    for k in range(4):
        pltpu.sync_copy(nr_hbm.at[pl.ds(k * _NTOR, _NTOR)], idx_buf)

        def gat_grp(g, _):
            iv = idx_buf[pl.ds(g * _L, _L)]
            ob0[pl.ds(g * _L, _L)] = plsc.load_gather(q0, [iv])
            ob1[pl.ds(g * _L, _L)] = plsc.load_gather(q1, [iv])
            return 0

        lax.fori_loop(0, _NTOR // _L, gat_grp, 0)
        pltpu.sync_copy(ob0, out_hbm.at[pl.ds((DIM * k + fbase) * _NTOR, _NTOR)])
        pltpu.sync_copy(ob1, out_hbm.at[pl.ds((DIM * k + fbase + 1) * _NTOR, _NTOR)])


@jax.jit
def _sc_gather(pa_flat, nr_flat):
    k = pl.kernel(
        _gat_body,
        out_type=jax.ShapeDtypeStruct((4 * DIM * _NTOR,), jnp.float32),
        mesh=_SC_MESH,
        scratch_types=[
            pltpu.VMEM((N_NODES,), jnp.float32),
            pltpu.VMEM((N_NODES,), jnp.float32),
            pltpu.VMEM((_NTOR,), jnp.int32),
            pltpu.VMEM((_NTOR,), jnp.float32),
            pltpu.VMEM((_NTOR,), jnp.float32),
            pltpu.SemaphoreType.DMA,
        ],
        compiler_params=_SC_PARAMS,
    )
    return k(pa_flat, nr_flat)


def _mpnn(p, xt_pad, src, dst, eat_pad, degp):
    w0t = jnp.zeros((DIM, 8), jnp.float32).at[:, 0:POINT_DIM].set(p['lin0_W'].T)
    b0 = p['lin0_b'][:, None]
    w1t = jnp.zeros((DIM, 8), jnp.float32).at[:, 0:EDGE_DIM].set(p['e_W1'].T)
    b1 = p['e_b1'][:, None]
    w2t = p['e_W2'].T
    b2 = p['e_b2'][:, None]
    mwt = p['m_W'].T
    mb = p['m_b'][:, None]
    wiht = p['g_Wih'].T
    whht = p['g_Whh'].T
    bih = p['g_bih'][:, None]
    bhh = p['g_bhh'][:, None]

    out0t, pt, deg = _tc_init(xt_pad, w0t, b0, mwt, mb, degp)
    et_flat = _tc_emlp(eat_pad, w1t, b1, w2t, b2).reshape(-1)
    ht = out0t
    for _ in range(6):
        agg = _sc_msg(pt.reshape(-1), et_flat, src, dst).reshape(DIM, N_NODES)
        ht, pt = _tc_gru(agg, ht, deg, wiht, whht, bih, bhh, mwt, mb)
    return ht


def kernel(x, edge_attr, actor_params, critic_params, edge_index, batch, nonring, nrbidx):
    src = edge_index[0]
    dst = edge_index[1]
    xt_pad = jnp.zeros((8, N_NODES), jnp.float32).at[0:POINT_DIM].set(x.T)
    eat_pad = jnp.zeros((8, N_EDGES), jnp.float32).at[0:EDGE_DIM].set(edge_attr.T)
    batch_row = batch[None, :]
    degp = _sc_deg(dst)

    def s2s_of(params, outt):
        sp, mp = params['s2s'], params['mem']
        return _tc_s2s(outt, batch_row, sp['Wih'].T, sp['Whh'].T,
                       sp['bih'][:, None], sp['bhh'][:, None],
                       mp['Wih'].T, mp['bih'][:, None], mp['bhh'][:, None])

    outt_a = _mpnn(actor_params['mpnn'], xt_pad, src, dst, eat_pad, degp)
    hpt, cpt = s2s_of(actor_params, outt_a)
    outt_c = _mpnn(critic_params['mpnn'], xt_pad, src, dst, eat_pad, degp)
    hvt, cvt = s2s_of(critic_params, outt_c)

    gath = _sc_gather(outt_a.reshape(-1),
                      nonring.T.reshape(-1)).reshape(4 * DIM, _NTOR)
    gnt = jax.random.gumbel(jax.random.key(1234), (N_GRAPHS, TPG, ACTION_DIM),
                            jnp.float32).reshape(_NTOR, ACTION_DIM).T
    ap, cp_mlp = actor_params['mlp'], critic_params['mlp']
    act, lp, ent, vt = _tc_head(
        hpt, gath, ap['W1'].T, ap['b1'][:, None], ap['W2'].T, ap['b2'][:, None],
        hvt, cp_mlp['W1'].T, cp_mlp['b1'][:, None], cp_mlp['W2'].T,
        cp_mlp['b2'][:, None], gnt)

    action = act.reshape(N_GRAPHS, TPG)
    log_prob = lp.reshape(N_GRAPHS, TPG)
    entropy = ent.reshape(N_GRAPHS, TPG)
    v = vt.reshape(N_GRAPHS, 1)
    return (action, log_prob, entropy, v, hpt.T, cpt.T, hvt.T, cvt.T)


# speedup vs baseline: 1.5878x; 1.0632x over previous
import functools

import jax
import jax.numpy as jnp
from jax import lax
from jax.experimental import pallas as pl
from jax.experimental.pallas import tpu as pltpu
from jax.experimental.pallas import tpu_sc as plsc

N_NODES = 10000
N_EDGES = 320000
DIM = 64
EDGE_DIM = 7
POINT_DIM = 3
N_GRAPHS = 128
TPG = 16
ACTION_DIM = 36


_SC_INFO = plsc.get_sparse_core_info()
_NC, _NS, _L = _SC_INFO.num_cores, _SC_INFO.num_subcores, _SC_INFO.num_lanes
_NW = _NC * _NS
_SC_MESH = plsc.VectorSubcoreMesh(core_axis_name="c", subcore_axis_name="s")
_SC_PARAMS = pltpu.CompilerParams(needs_layout_passes=False)

_EPW = N_EDGES // _NW


def _deg_body(dst_hbm, out_hbm, dst_v, hist_v, sem):
    wid = lax.axis_index("s") * _NC + lax.axis_index("c")
    base = wid * _EPW
    pltpu.sync_copy(dst_hbm.at[pl.ds(base, _EPW)], dst_v)
    zeros = jnp.zeros((_L,), jnp.float32)
    ones = jnp.ones((_L,), jnp.float32)

    def zero_body(i, _):
        hist_v[pl.ds(i * _L, _L)] = zeros
        return 0

    lax.fori_loop(0, N_NODES // _L, zero_body, 0)

    def acc_body(g, _):
        dv = dst_v[pl.ds(g * _L, _L)]
        plsc.addupdate_scatter(hist_v, [dv], ones)
        return 0

    lax.fori_loop(0, _EPW // _L, acc_body, 0)
    pltpu.sync_copy(hist_v, out_hbm.at[wid])


@functools.partial(jax.jit, static_argnames=())
def _sc_deg(dst):
    k = pl.kernel(
        _deg_body,
        out_type=jax.ShapeDtypeStruct((_NW, N_NODES), jnp.float32),
        mesh=_SC_MESH,
        scratch_types=[
            pltpu.VMEM((_EPW,), jnp.int32),
            pltpu.VMEM((N_NODES,), jnp.float32),
            pltpu.SemaphoreType.DMA,
        ],
        compiler_params=_SC_PARAMS,
    )
    return k(dst)


_FPW = DIM // _NW
_MSG_CH = 6400
_NCHUNK = N_EDGES // _MSG_CH
_UNROLL = 8


def _msg_body(pt_hbm, et_hbm, src_hbm, dst_hbm, out_hbm, p0, p1, a0, a1,
              e0A, e0B, e1A, e1B, sA, sB, dA, dB, sem):
    wid = lax.axis_index("s") * _NC + lax.axis_index("c")
    fbase = wid * _FPW
    slots = ((e0A, e1A, sA, dA), (e0B, e1B, sB, dB))

    def start_chunk(c, slot):
        e0b, e1b, sb, db = slots[slot]
        ecp0 = pltpu.async_copy(
            et_hbm.at[pl.ds(fbase * N_EDGES + c * _MSG_CH, _MSG_CH)], e0b, sem)
        ecp1 = pltpu.async_copy(
            et_hbm.at[pl.ds((fbase + 1) * N_EDGES + c * _MSG_CH, _MSG_CH)], e1b, sem)
        scp = pltpu.async_copy(src_hbm.at[pl.ds(c * _MSG_CH, _MSG_CH)], sb, sem)
        dcp = pltpu.async_copy(dst_hbm.at[pl.ds(c * _MSG_CH, _MSG_CH)], db, sem)
        return ecp0, ecp1, scp, dcp

    cps0 = start_chunk(0, 0)
    pltpu.sync_copy(pt_hbm.at[pl.ds(fbase * N_NODES, N_NODES)], p0)
    pltpu.sync_copy(pt_hbm.at[pl.ds((fbase + 1) * N_NODES, N_NODES)], p1)
    zeros = jnp.zeros((_L,), jnp.float32)

    def zero_body(i, _):
        for u in range(5):
            a0[pl.ds((i * 5 + u) * _L, _L)] = zeros
            a1[pl.ds((i * 5 + u) * _L, _L)] = zeros
        return 0

    lax.fori_loop(0, N_NODES // (_L * 5), zero_body, 0)

    def do_chunk(slot):
        e0b, e1b, sb, db = slots[slot]

        def grp_body(i, _):
            msgs = []
            for u in range(_UNROLL):
                g = i * _UNROLL + u
                sv = sb[pl.ds(g * _L, _L)]
                dv = db[pl.ds(g * _L, _L)]
                r0 = plsc.load_gather(p0, [sv])
                e0 = e0b[pl.ds(g * _L, _L)]
                r1 = plsc.load_gather(p1, [sv])
                e1 = e1b[pl.ds(g * _L, _L)]
                msgs.append((dv, jnp.maximum(r0 + e0, 0.0), jnp.maximum(r1 + e1, 0.0)))
            for dv, m0, m1 in msgs:
                plsc.addupdate_scatter(a0, [dv], m0)
                plsc.addupdate_scatter(a1, [dv], m1)
            return 0

        lax.fori_loop(0, _MSG_CH // (_L * _UNROLL), grp_body, 0)

    cps = cps0
    for c in range(_NCHUNK):
        for cp in cps:
            cp.wait()
        if c + 1 < _NCHUNK:
            cps = start_chunk(c + 1, (c + 1) % 2)
        do_chunk(c % 2)

    pltpu.sync_copy(a0, out_hbm.at[pl.ds(fbase * N_NODES, N_NODES)])
    pltpu.sync_copy(a1, out_hbm.at[pl.ds((fbase + 1) * N_NODES, N_NODES)])


@jax.jit
def _sc_msg(pt_flat, et_flat, src, dst):
    k = pl.kernel(
        _msg_body,
        out_type=jax.ShapeDtypeStruct((DIM * N_NODES,), jnp.float32),
        mesh=_SC_MESH,
        scratch_types=[
            pltpu.VMEM((N_NODES,), jnp.float32),
            pltpu.VMEM((N_NODES,), jnp.float32),
            pltpu.VMEM((N_NODES,), jnp.float32),
            pltpu.VMEM((N_NODES,), jnp.float32),
            pltpu.VMEM((_MSG_CH,), jnp.float32),
            pltpu.VMEM((_MSG_CH,), jnp.float32),
            pltpu.VMEM((_MSG_CH,), jnp.float32),
            pltpu.VMEM((_MSG_CH,), jnp.float32),
            pltpu.VMEM((_MSG_CH,), jnp.int32),
            pltpu.VMEM((_MSG_CH,), jnp.int32),
            pltpu.VMEM((_MSG_CH,), jnp.int32),
            pltpu.VMEM((_MSG_CH,), jnp.int32),
            pltpu.SemaphoreType.DMA,
        ],
        compiler_params=_SC_PARAMS,
    )
    return k(pt_flat, et_flat, src, dst)


def _init_body(xt_ref, w0t_ref, b0_ref, mwt_ref, mb_ref, degp_ref,
               out0_ref, pt0_ref, deg_ref):
    out0 = jnp.maximum(jnp.dot(w0t_ref[...], xt_ref[...],
                               preferred_element_type=jnp.float32) + b0_ref[...], 0.0)
    out0_ref[...] = out0
    pt0_ref[...] = jnp.dot(mwt_ref[...], out0,
                           preferred_element_type=jnp.float32) + mb_ref[...]
    deg_ref[...] = jnp.maximum(jnp.sum(degp_ref[...], axis=0, keepdims=True), 1.0)


@jax.jit
def _tc_init(xt, w0t, b0, mwt, mb, degp):
    return pl.pallas_call(
        _init_body,
        out_shape=(
            jax.ShapeDtypeStruct((DIM, N_NODES), jnp.float32),
            jax.ShapeDtypeStruct((DIM, N_NODES), jnp.float32),
            jax.ShapeDtypeStruct((1, N_NODES), jnp.float32),
        ),
    )(xt, w0t, b0, mwt, mb, degp)


_GRU_BLK = 2500


def _gru_body(aggt_ref, ht_ref, deg_ref, wiht_ref, whht_ref, bih_ref, bhh_ref,
              mwt_ref, mb_ref, h2_ref, pt2_ref):
    aggn = aggt_ref[...] / deg_ref[...]
    gi = jnp.dot(wiht_ref[...], aggn, preferred_element_type=jnp.float32) + bih_ref[...]
    gh = jnp.dot(whht_ref[...], ht_ref[...], preferred_element_type=jnp.float32) + bhh_ref[...]
    r = jax.nn.sigmoid(gi[0:DIM] + gh[0:DIM])
    z = jax.nn.sigmoid(gi[DIM:2 * DIM] + gh[DIM:2 * DIM])
    n = jnp.tanh(gi[2 * DIM:3 * DIM] + r * gh[2 * DIM:3 * DIM])
    h2 = (1.0 - z) * n + z * ht_ref[...]
    h2_ref[...] = h2
    pt2_ref[...] = jnp.dot(mwt_ref[...], h2, preferred_element_type=jnp.float32) + mb_ref[...]


@jax.jit
def _tc_gru(aggt, ht, deg, wiht, whht, bih, bhh, mwt, mb):
    return pl.pallas_call(
        _gru_body,
        out_shape=(
            jax.ShapeDtypeStruct((DIM, N_NODES), jnp.float32),
            jax.ShapeDtypeStruct((DIM, N_NODES), jnp.float32),
        ),
    )(aggt, ht, deg, wiht, whht, bih, bhh, mwt, mb)


_EMLP_BLK = 3200


def _emlp_body(eat_ref, w1t_ref, b1_ref, w2t_ref, b2_ref, et_ref):
    h1 = jnp.maximum(jnp.dot(w1t_ref[...], eat_ref[...],
                             preferred_element_type=jnp.float32) + b1_ref[...], 0.0)
    et_ref[...] = jnp.dot(w2t_ref[...], h1,
                          preferred_element_type=jnp.float32) + b2_ref[...]


@jax.jit
def _tc_emlp(eat, w1t, b1, w2t, b2):
    nb = N_EDGES // _EMLP_BLK
    full = lambda s: pl.BlockSpec(s, lambda i: (0, 0))
    col = lambda r: pl.BlockSpec((r, _EMLP_BLK), lambda i: (0, i))
    return pl.pallas_call(
        _emlp_body,
        grid=(nb,),
        in_specs=[col(8), full((DIM, 8)), full((DIM, 1)),
                  full((DIM, DIM)), full((DIM, 1))],
        out_specs=col(DIM),
        out_shape=jax.ShapeDtypeStruct((DIM, N_EDGES), jnp.float32),
    )(eat, w1t, b1, w2t, b2)


def _s2s_body(outt_ref, batch_ref, wih_ref, whh_ref, bih_ref, bhh_ref,
              wihm_ref, bihm_ref, bhhm_ref, hp_ref, cp_ref):
    outt = outt_ref[...]
    bt = batch_ref[...]
    giota = lax.broadcasted_iota(jnp.int32, (N_GRAPHS, 1), 0)
    B = (bt == giota).astype(jnp.float32)
    wih = wih_ref[...]
    whh = whh_ref[...]
    bih = bih_ref[...]
    bhh = bhh_ref[...]
    ht = jnp.zeros((DIM, N_GRAPHS), jnp.float32)
    ct = jnp.zeros((DIM, N_GRAPHS), jnp.float32)
    qt = jnp.zeros((2 * DIM, N_GRAPHS), jnp.float32)
    for _ in range(6):
        g = jnp.dot(wih, qt, preferred_element_type=jnp.float32) \
            + jnp.dot(whh, ht, preferred_element_type=jnp.float32) + bih + bhh
        ii = jax.nn.sigmoid(g[0:DIM])
        ff = jax.nn.sigmoid(g[DIM:2 * DIM])
        gg = jnp.tanh(g[2 * DIM:3 * DIM])
        oo = jax.nn.sigmoid(g[3 * DIM:4 * DIM])
        ct = ff * ct + ii * gg
        ht = oo * jnp.tanh(ct)
        hn = jnp.dot(ht, B, preferred_element_type=jnp.float32)
        e = jnp.sum(outt * hn, axis=0, keepdims=True)
        emax = jnp.max(jnp.where(B > 0.0, e, -1e30), axis=1, keepdims=True)
        emax_n = jnp.dot(emax.reshape(1, N_GRAPHS), B,
                         preferred_element_type=jnp.float32)
        ex = jnp.exp(e - emax_n)
        den = jnp.sum(B * ex, axis=1, keepdims=True)
        den_n = jnp.dot(den.reshape(1, N_GRAPHS), B,
                        preferred_element_type=jnp.float32)
        a = ex / (den_n + 1e-16)
        rt = jnp.dot(outt * a, B.T, preferred_element_type=jnp.float32)
        qt = jnp.concatenate([ht, rt], axis=0)
    gm = jnp.dot(wihm_ref[...], qt, preferred_element_type=jnp.float32) \
        + bihm_ref[...] + bhhm_ref[...]
    im = jax.nn.sigmoid(gm[0:DIM])
    gm2 = jnp.tanh(gm[2 * DIM:3 * DIM])
    om = jax.nn.sigmoid(gm[3 * DIM:4 * DIM])
    cpt = im * gm2
    hp_ref[...] = om * jnp.tanh(cpt)
    cp_ref[...] = cpt


@jax.jit
def _tc_s2s(outt, batch_row, wihT, whhT, bih, bhh, wihmT, bihm, bhhm):
    return pl.pallas_call(
        _s2s_body,
        out_shape=(
            jax.ShapeDtypeStruct((DIM, N_GRAPHS), jnp.float32),
            jax.ShapeDtypeStruct((DIM, N_GRAPHS), jnp.float32),
        ),
    )(outt, batch_row, wihT, whhT, bih, bhh, wihmT, bihm, bhhm)


_NTOR = N_GRAPHS * TPG


def _head_body(hpt_ref, gath_ref, w1_ref, b1_ref, w2_ref, b2_ref,
               hvt_ref, w1c_ref, b1c_ref, w2c_ref, b2c_ref, gn_ref,
               act_ref, lp_ref, ent_ref, v_ref):
    giota = lax.broadcasted_iota(jnp.int32, (N_GRAPHS, 1), 0)
    jg = lax.broadcasted_iota(jnp.int32, (1, _NTOR), 1) // TPG
    R = (jg == giota).astype(jnp.float32)
    sel = jnp.dot(hpt_ref[...], R, preferred_element_type=jnp.float32)
    cat = jnp.concatenate([sel, gath_ref[...]], axis=0)
    h1 = jnp.maximum(jnp.dot(w1_ref[...], cat,
                             preferred_element_type=jnp.float32) + b1_ref[...], 0.0)
    logits = jnp.dot(w2_ref[...], h1,
                     preferred_element_type=jnp.float32) + b2_ref[...]
    m = jnp.max(logits, axis=0, keepdims=True)
    sh = logits - m
    lse = jnp.log(jnp.sum(jnp.exp(sh), axis=0, keepdims=True))
    logp = sh - lse
    per = gn_ref[...] + logits
    pm = jnp.max(per, axis=0, keepdims=True)
    riota = lax.broadcasted_iota(jnp.int32, (ACTION_DIM, _NTOR), 0)
    act = jnp.min(jnp.where(per == pm, riota, ACTION_DIM), axis=0, keepdims=True)
    act_ref[...] = act
    lp_ref[...] = jnp.sum(jnp.where(riota == act, logp, 0.0), axis=0, keepdims=True)
    ent_ref[...] = -jnp.sum(jnp.exp(logp) * logp, axis=0, keepdims=True)
    h1c = jnp.maximum(jnp.dot(w1c_ref[...], hvt_ref[...],
                              preferred_element_type=jnp.float32) + b1c_ref[...], 0.0)
    v_ref[...] = jnp.dot(w2c_ref[...], h1c,
                         preferred_element_type=jnp.float32) + b2c_ref[...]


@jax.jit
def _tc_head(hpt, gath, w1, b1, w2, b2, hvt, w1c, b1c, w2c, b2c, gn):
    return pl.pallas_call(
        _head_body,
        out_shape=(
            jax.ShapeDtypeStruct((1, _NTOR), jnp.int32),
            jax.ShapeDtypeStruct((1, _NTOR), jnp.float32),
            jax.ShapeDtypeStruct((1, _NTOR), jnp.float32),
            jax.ShapeDtypeStruct((1, N_GRAPHS), jnp.float32),
        ),
    )(hpt, gath, w1, b1, w2, b2, hvt, w1c, b1c, w2c, b2c, gn)


def _gat_body(pa_hbm, nr_hbm, out_hbm, q0, q1, idx_buf, ob0, ob1, sem):
    wid = lax.axis_index("s") * _NC + lax.axis_index("c")
    fbase = wid * _FPW
    pltpu.sync_copy(pa_hbm.at[pl.ds(fbase * N_NODES, N_NODES)], q0)
    pltpu.sync_copy(pa_hbm.at[pl.ds((fbase + 1) * N_NODES, N_NODES)], q1)
    for k in range(4):
        pltpu.sync_copy(nr_hbm.at[pl.ds(k * _NTOR, _NTOR)], idx_buf)

        def gat_grp(g, _):
            iv = idx_buf[pl.ds(g * _L, _L)]
            ob0[pl.ds(g * _L, _L)] = plsc.load_gather(q0, [iv])
            ob1[pl.ds(g * _L, _L)] = plsc.load_gather(q1, [iv])
            return 0

        lax.fori_loop(0, _NTOR // _L, gat_grp, 0)
        pltpu.sync_copy(ob0, out_hbm.at[pl.ds((DIM * k + fbase) * _NTOR, _NTOR)])
        pltpu.sync_copy(ob1, out_hbm.at[pl.ds((DIM * k + fbase + 1) * _NTOR, _NTOR)])


@jax.jit
def _sc_gather(pa_flat, nr_flat):
    k = pl.kernel(
        _gat_body,
        out_type=jax.ShapeDtypeStruct((4 * DIM * _NTOR,), jnp.float32),
        mesh=_SC_MESH,
        scratch_types=[
            pltpu.VMEM((N_NODES,), jnp.float32),
            pltpu.VMEM((N_NODES,), jnp.float32),
            pltpu.VMEM((_NTOR,), jnp.int32),
            pltpu.VMEM((_NTOR,), jnp.float32),
            pltpu.VMEM((_NTOR,), jnp.float32),
            pltpu.SemaphoreType.DMA,
        ],
        compiler_params=_SC_PARAMS,
    )
    return k(pa_flat, nr_flat)


def _gru_cell(x, h, Wih, Whh, bih, bhh):
    gi = x @ Wih + bih
    gh = h @ Whh + bhh
    ir, iz, inn = jnp.split(gi, 3, axis=-1)
    hr, hz, hn = jnp.split(gh, 3, axis=-1)
    r = jax.nn.sigmoid(ir + hr)
    z = jax.nn.sigmoid(iz + hz)
    n = jnp.tanh(inn + r * hn)
    return (1.0 - z) * n + z * h


def _mpnn(p, x, src, dst, edge_attr, degp):
    out = jax.nn.relu(x @ p['lin0_W'] + p['lin0_b'])
    e = jax.nn.relu(edge_attr @ p['e_W1'] + p['e_b1']) @ p['e_W2'] + p['e_b2']
    deg = jnp.maximum(jnp.sum(degp, axis=0), 1.0)[:, None]
    eT = e.T.reshape(-1)
    h = out
    for _ in range(6):
        pt = (out @ p['m_W'] + p['m_b']).T.reshape(-1)
        agg = _sc_msg(pt, eT, src, dst).reshape(DIM, N_NODES).T / deg
        h = _gru_cell(agg, h, p['g_Wih'], p['g_Whh'], p['g_bih'], p['g_bhh'])
        out = h
    return out.T


def _lstm_cell(x, h, c, Wih, Whh, bih, bhh):
    g = x @ Wih + h @ Whh + bih + bhh
    i, f, gg, o = jnp.split(g, 4, axis=-1)
    i = jax.nn.sigmoid(i)
    f = jax.nn.sigmoid(f)
    gg = jnp.tanh(gg)
    o = jax.nn.sigmoid(o)
    c2 = f * c + i * gg
    h2 = o * jnp.tanh(c2)
    return h2, c2


def _set2set(p, out, batch):
    h = jnp.zeros((N_GRAPHS, DIM), jnp.float32)
    c = jnp.zeros((N_GRAPHS, DIM), jnp.float32)
    q_star = jnp.zeros((N_GRAPHS, 2 * DIM), jnp.float32)
    for _ in range(6):
        h, c = _lstm_cell(q_star, h, c, p['Wih'], p['Whh'], p['bih'], p['bhh'])
        e = jnp.sum(out * h[batch], axis=-1)
        emax = jax.ops.segment_max(e, batch, num_segments=N_GRAPHS)
        ex = jnp.exp(e - emax[batch])
        den = jax.ops.segment_sum(ex, batch, num_segments=N_GRAPHS)
        a = ex / (den[batch] + 1e-16)
        r = jax.ops.segment_sum(a[:, None] * out, batch, num_segments=N_GRAPHS)
        q_star = jnp.concatenate([h, r], axis=-1)
    return q_star


def kernel(x, edge_attr, actor_params, critic_params, edge_index, batch, nonring, nrbidx):
    src = edge_index[0]
    dst = edge_index[1]
    h0 = jnp.zeros((N_GRAPHS, DIM), jnp.float32)
    c0 = jnp.zeros((N_GRAPHS, DIM), jnp.float32)
    degp = _sc_deg(dst)

    outt_a = _mpnn(actor_params['mpnn'], x, src, dst, edge_attr, degp)
    out_a = outt_a.T
    pool_a = _set2set(actor_params['s2s'], out_a, batch)
    mp = actor_params['mem']
    hp, cp = _lstm_cell(pool_a, h0, c0, mp['Wih'], mp['Whh'], mp['bih'], mp['bhh'])
    lstm_sel = hp[nrbidx]
    gath = _sc_gather(outt_a.reshape(-1),
                      nonring.T.reshape(-1)).reshape(4 * DIM, _NTOR).T
    cat = jnp.concatenate([lstm_sel, gath], axis=1)
    ap = actor_params['mlp']
    logits = (jax.nn.relu(cat @ ap['W1'] + ap['b1']) @ ap['W2'] + ap['b2']).reshape(N_GRAPHS, TPG, ACTION_DIM)
    outt_c = _mpnn(critic_params['mpnn'], x, src, dst, edge_attr, degp)
    out_c = outt_c.T
    pool_c = _set2set(critic_params['s2s'], out_c, batch)
    mc = critic_params['mem']
    hv, cv = _lstm_cell(pool_c, h0, c0, mc['Wih'], mc['Whh'], mc['bih'], mc['bhh'])
    cpp = critic_params['mlp']
    v = jax.nn.relu(hv @ cpp['W1'] + cpp['b1']) @ cpp['W2'] + cpp['b2']
    gnoise = jax.random.gumbel(jax.random.key(1234), (N_GRAPHS, TPG, ACTION_DIM), jnp.float32)
    logp_all = jax.nn.log_softmax(logits, axis=-1)
    action = jnp.argmax(gnoise + logits, axis=-1)
    log_prob = jnp.take_along_axis(logp_all, action[..., None], axis=-1)[..., 0]
    entropy = -jnp.sum(jnp.exp(logp_all) * logp_all, axis=-1)
    return (action, log_prob, entropy, v, hp, cp, hv, cv)
